# baseline (speedup 1.0000x reference)
import jax
import jax.numpy as jnp
from jax.experimental import pallas as pl
from jax.experimental.pallas import tpu as pltpu

_MAX_NUM_TILES = 4
_NUM_PATCHES = 1601
_HIDDEN = 1280
_P_BLK = 512


def _body(ids_ref, gate_ref, hs_ref, emb_ref, tile_ref, out_ref):
    g = jnp.tanh(gate_ref[0])
    out_ref[...] = hs_ref[...] + (1.0 - g) * emb_ref[...] + g * tile_ref[...]


def kernel(hidden_state, gate, embedding, tile_embedding, aspect_ratio_ids):
    b, t, np_, h = hidden_state.shape
    n_pb = pl.cdiv(np_, _P_BLK)
    tile4 = tile_embedding.reshape(tile_embedding.shape[0], t, np_, h)

    grid_spec = pltpu.PrefetchScalarGridSpec(
        num_scalar_prefetch=2,
        grid=(n_pb, b, t),
        in_specs=[
            pl.BlockSpec((1, 1, _P_BLK, h), lambda p, i, j, ids, g: (i, j, p, 0)),
            pl.BlockSpec((_P_BLK, h), lambda p, i, j, ids, g: (p, 0)),
            pl.BlockSpec((1, 1, _P_BLK, h), lambda p, i, j, ids, g: (ids[i], j, p, 0)),
        ],
        out_specs=pl.BlockSpec((1, 1, _P_BLK, h), lambda p, i, j, ids, g: (i, j, p, 0)),
    )
    return pl.pallas_call(
        _body,
        grid_spec=grid_spec,
        out_shape=jax.ShapeDtypeStruct(hidden_state.shape, hidden_state.dtype),
        compiler_params=pltpu.CompilerParams(
            dimension_semantics=("arbitrary", "arbitrary", "arbitrary"),
        ),
    )(aspect_ratio_ids, gate, hidden_state, embedding, tile4)


# speedup vs baseline: 13.5636x; 13.5636x over previous
import functools

import jax
import jax.numpy as jnp
from jax.experimental import pallas as pl
from jax.experimental.pallas import tpu as pltpu

_PB = 256


def _body(ids_ref, gate_ref, hs_ref, emb_ref, tile_hbm, out_ref, scratch, sems,
          *, np_, t, h, pb, n_pb, nb):
    bi = pl.program_id(0)
    p = pl.program_id(1)
    step = bi * n_pb + p
    chunk = pb * h
    tstride = np_ * h
    last = n_pb - 1
    tail_rows = np_ - last * pb
    tail_chunk = tail_rows * h

    def issue(bi_, p_, slot):
        row = ids_ref[bi_]

        @pl.when(p_ < last)
        def _():
            for tau in range(t):
                pltpu.make_async_copy(
                    tile_hbm.at[pl.ds(row, 1), pl.ds(tau * tstride + p_ * chunk, chunk)],
                    scratch.at[slot, tau],
                    sems.at[slot, tau],
                ).start()

        @pl.when(p_ == last)
        def _():
            for tau in range(t):
                pltpu.make_async_copy(
                    tile_hbm.at[pl.ds(row, 1), pl.ds(tau * tstride + last * chunk, tail_chunk)],
                    scratch.at[slot, tau, pl.ds(0, 1), pl.ds(0, tail_chunk)],
                    sems.at[slot, tau],
                ).start()

    slot = jax.lax.rem(step, 2)

    @pl.when(step == 0)
    def _():
        issue(bi, p, 0)

    @pl.when(step + 1 < nb * n_pb)
    def _():
        nxt_p = jax.lax.rem(p + 1, n_pb)
        nxt_b = bi + jnp.where(p + 1 == n_pb, 1, 0)
        issue(nxt_b, nxt_p, 1 - slot)

    @pl.when(p < last)
    def _():
        for tau in range(t):
            pltpu.make_async_copy(
                tile_hbm.at[pl.ds(0, 1), pl.ds(0, chunk)],
                scratch.at[slot, tau],
                sems.at[slot, tau],
            ).wait()

    @pl.when(p == last)
    def _():
        for tau in range(t):
            pltpu.make_async_copy(
                tile_hbm.at[pl.ds(0, 1), pl.ds(0, tail_chunk)],
                scratch.at[slot, tau, pl.ds(0, 1), pl.ds(0, tail_chunk)],
                sems.at[slot, tau],
            ).wait()

    g = jnp.tanh(gate_ref[0])
    pos = (1.0 - g) * emb_ref[...]
    base = hs_ref[...]
    for tau in range(t):
        tile_v = scratch[slot, tau, 0].reshape(pb, h)
        out_ref[0, :, tau, :] = base[0, :, tau, :] + pos + g * tile_v


def kernel(hidden_state, gate, embedding, tile_embedding, aspect_ratio_ids):
    b, t, np_, h = hidden_state.shape
    hs_t = jnp.transpose(hidden_state, (0, 2, 1, 3))
    n_pb = pl.cdiv(np_, _PB)

    body = functools.partial(_body, np_=np_, t=t, h=h, pb=_PB, n_pb=n_pb, nb=b)

    grid_spec = pltpu.PrefetchScalarGridSpec(
        num_scalar_prefetch=2,
        grid=(b, n_pb),
        in_specs=[
            pl.BlockSpec((1, _PB, t, h), lambda i, p, ids, g: (i, p, 0, 0)),
            pl.BlockSpec((_PB, h), lambda i, p, ids, g: (p, 0)),
            pl.BlockSpec(memory_space=pltpu.MemorySpace.HBM),
        ],
        out_specs=pl.BlockSpec((1, _PB, t, h), lambda i, p, ids, g: (i, p, 0, 0)),
        scratch_shapes=[
            pltpu.VMEM((2, t, 1, _PB * h), jnp.float32),
            pltpu.SemaphoreType.DMA((2, t)),
        ],
    )
    out_t = pl.pallas_call(
        body,
        grid_spec=grid_spec,
        out_shape=jax.ShapeDtypeStruct((b, np_, t, h), hidden_state.dtype),
        compiler_params=pltpu.CompilerParams(
            dimension_semantics=("arbitrary", "arbitrary"),
        ),
    )(aspect_ratio_ids, gate, hs_t, embedding, tile_embedding)
    return jnp.transpose(out_t, (0, 2, 1, 3))


# speedup vs baseline: 23.9778x; 1.7678x over previous
import functools

import jax
import jax.numpy as jnp
from jax.experimental import pallas as pl
from jax.experimental.pallas import tpu as pltpu
from jax._src.state.indexing import Slice as _Slice

_PB = 256


def _body(ids_ref, gate_ref, hs_ref, emb_ref, tile_hbm, out_ref, scratch, sems,
          *, np_, t, h, pb, n_pb, nb):
    bi = pl.program_id(0)
    p = pl.program_id(1)
    step = bi * n_pb + p
    chunk = pb * h
    nj = h // 128
    tstride = np_ * h
    last = n_pb - 1
    tail_rows = np_ - last * pb
    tail_chunk = tail_rows * h

    def issue(bi_, p_, slot):
        row = ids_ref[bi_]

        @pl.when(p_ < last)
        def _():
            for tau in range(t):
                pltpu.make_async_copy(
                    tile_hbm.at[pl.ds(row, 1), pl.ds(tau * tstride + p_ * chunk, chunk)],
                    scratch.at[slot, tau].reshape(1, chunk),
                    sems.at[slot, tau],
                ).start()

        @pl.when(p_ == last)
        def _():
            for tau in range(t):
                pltpu.make_async_copy(
                    tile_hbm.at[pl.ds(row, 1), pl.ds(tau * tstride + last * chunk, tail_chunk)],
                    scratch.at[slot, tau, pl.ds(0, tail_chunk // 128), :].reshape(1, tail_chunk),
                    sems.at[slot, tau],
                ).start()

    slot = jax.lax.rem(step, 2)

    @pl.when(step == 0)
    def _():
        issue(bi, p, 0)

    @pl.when(step + 1 < nb * n_pb)
    def _():
        nxt_p = jax.lax.rem(p + 1, n_pb)
        nxt_b = bi + jnp.where(p + 1 == n_pb, 1, 0)
        issue(nxt_b, nxt_p, 1 - slot)

    @pl.when(p < last)
    def _():
        for tau in range(t):
            pltpu.make_async_copy(
                tile_hbm.at[pl.ds(0, 1), pl.ds(0, chunk)],
                scratch.at[slot, tau].reshape(1, chunk),
                sems.at[slot, tau],
            ).wait()

    @pl.when(p == last)
    def _():
        for tau in range(t):
            pltpu.make_async_copy(
                tile_hbm.at[pl.ds(0, 1), pl.ds(0, tail_chunk)],
                scratch.at[slot, tau, pl.ds(0, tail_chunk // 128), :].reshape(1, tail_chunk),
                sems.at[slot, tau],
            ).wait()

    g = jnp.tanh(gate_ref[0])
    pos = (1.0 - g) * emb_ref[...]
    for tau in range(t):
        tile_v = jnp.concatenate(
            [scratch[slot, tau, _Slice(j, pb, nj), :] for j in range(nj)], axis=1
        )
        base_t = hs_ref[0, :, tau, :]
        out_ref[0, :, tau, :] = base_t + pos + g * tile_v


def kernel(hidden_state, gate, embedding, tile_embedding, aspect_ratio_ids):
    b, t, np_, h = hidden_state.shape
    hs_t = jnp.transpose(hidden_state, (0, 2, 1, 3))
    n_pb = pl.cdiv(np_, _PB)

    body = functools.partial(_body, np_=np_, t=t, h=h, pb=_PB, n_pb=n_pb, nb=b)

    grid_spec = pltpu.PrefetchScalarGridSpec(
        num_scalar_prefetch=2,
        grid=(b, n_pb),
        in_specs=[
            pl.BlockSpec((1, _PB, t, h), lambda i, p, ids, g: (i, p, 0, 0)),
            pl.BlockSpec((_PB, h), lambda i, p, ids, g: (p, 0)),
            pl.BlockSpec(memory_space=pltpu.MemorySpace.HBM),
        ],
        out_specs=pl.BlockSpec((1, _PB, t, h), lambda i, p, ids, g: (i, p, 0, 0)),
        scratch_shapes=[
            pltpu.VMEM((2, t, _PB * h // 128, 128), jnp.float32),
            pltpu.SemaphoreType.DMA((2, t)),
        ],
    )
    out_t = pl.pallas_call(
        body,
        grid_spec=grid_spec,
        out_shape=jax.ShapeDtypeStruct((b, np_, t, h), hidden_state.dtype),
        compiler_params=pltpu.CompilerParams(
            dimension_semantics=("arbitrary", "arbitrary"),
        ),
    )(aspect_ratio_ids, gate, hs_t, embedding, tile_embedding)
    return jnp.transpose(out_t, (0, 2, 1, 3))


# speedup vs baseline: 25.9791x; 1.0835x over previous
import functools

import jax
import jax.numpy as jnp
from jax.experimental import pallas as pl
from jax.experimental.pallas import tpu as pltpu
from jax._src.state.indexing import Slice as _Slice

_PB = 256


def _body(ids_ref, perm_ref, gate_ref, hs_ref, emb_ref, tile_hbm, out_ref,
          scratch, sems, state, *, np_, t, h, pb, n_pb, nb):
    p = pl.program_id(0)
    k = pl.program_id(1)
    step = p * nb + k
    chunk = pb * h
    nj = h // 128
    tstride = np_ * h
    last = n_pb - 1
    tail_chunk = (np_ - last * pb) * h

    def issue(p_, k_, slot):
        row = ids_ref[perm_ref[k_]]

        @pl.when(p_ < last)
        def _():
            for tau in range(t):
                pltpu.make_async_copy(
                    tile_hbm.at[pl.ds(row, 1), pl.ds(tau * tstride + p_ * chunk, chunk)],
                    scratch.at[slot, tau].reshape(1, chunk),
                    sems.at[slot, tau],
                ).start()

        @pl.when(p_ == last)
        def _():
            for tau in range(t):
                pltpu.make_async_copy(
                    tile_hbm.at[pl.ds(row, 1), pl.ds(tau * tstride + last * chunk, tail_chunk)],
                    scratch.at[slot, tau, pl.ds(0, tail_chunk // 128), :].reshape(1, tail_chunk),
                    sems.at[slot, tau],
                ).start()

    first = step == 0
    my_slot = jnp.where(first, 0, state[0])
    fresh = jnp.where(first, 1, state[1])

    @pl.when(first)
    def _():
        issue(p, k, 0)

    nxt_k = jax.lax.rem(k + 1, nb)
    nxt_p = p + jnp.where(k + 1 == nb, 1, 0)
    valid = nxt_p < n_pb
    dup = jnp.logical_and(nxt_k != 0, ids_ref[perm_ref[nxt_k]] == ids_ref[perm_ref[k]])
    nxt_slot = jnp.where(dup, my_slot, 1 - my_slot)

    @pl.when(jnp.logical_and(valid, jnp.logical_not(dup)))
    def _():
        issue(nxt_p, nxt_k, nxt_slot)

    state[0] = nxt_slot
    state[1] = jnp.where(dup, 0, 1).astype(jnp.int32)

    @pl.when(jnp.logical_and(fresh == 1, p < last))
    def _():
        for tau in range(t):
            pltpu.make_async_copy(
                tile_hbm.at[pl.ds(0, 1), pl.ds(0, chunk)],
                scratch.at[my_slot, tau].reshape(1, chunk),
                sems.at[my_slot, tau],
            ).wait()

    @pl.when(jnp.logical_and(fresh == 1, p == last))
    def _():
        for tau in range(t):
            pltpu.make_async_copy(
                tile_hbm.at[pl.ds(0, 1), pl.ds(0, tail_chunk)],
                scratch.at[my_slot, tau, pl.ds(0, tail_chunk // 128), :].reshape(1, tail_chunk),
                sems.at[my_slot, tau],
            ).wait()

    g = jnp.tanh(gate_ref[0])
    pos = (1.0 - g) * emb_ref[...]
    for tau in range(t):
        tile_v = jnp.concatenate(
            [scratch[my_slot, tau, _Slice(j, pb, nj), :] for j in range(nj)], axis=1
        )
        base_t = hs_ref[0, :, tau, :]
        out_ref[0, :, tau, :] = base_t + pos + g * tile_v


def kernel(hidden_state, gate, embedding, tile_embedding, aspect_ratio_ids):
    b, t, np_, h = hidden_state.shape
    hs_t = jnp.transpose(hidden_state, (0, 2, 1, 3))
    n_pb = pl.cdiv(np_, _PB)
    perm = jnp.argsort(aspect_ratio_ids).astype(jnp.int32)

    body = functools.partial(_body, np_=np_, t=t, h=h, pb=_PB, n_pb=n_pb, nb=b)

    grid_spec = pltpu.PrefetchScalarGridSpec(
        num_scalar_prefetch=3,
        grid=(n_pb, b),
        in_specs=[
            pl.BlockSpec((1, _PB, t, h), lambda p, k, ids, pr, g: (pr[k], p, 0, 0)),
            pl.BlockSpec((_PB, h), lambda p, k, ids, pr, g: (p, 0)),
            pl.BlockSpec(memory_space=pltpu.MemorySpace.HBM),
        ],
        out_specs=pl.BlockSpec((1, _PB, t, h), lambda p, k, ids, pr, g: (pr[k], p, 0, 0)),
        scratch_shapes=[
            pltpu.VMEM((2, t, _PB * h // 128, 128), jnp.float32),
            pltpu.SemaphoreType.DMA((2, t)),
            pltpu.SMEM((2,), jnp.int32),
        ],
    )
    out_t = pl.pallas_call(
        body,
        grid_spec=grid_spec,
        out_shape=jax.ShapeDtypeStruct((b, np_, t, h), hidden_state.dtype),
        compiler_params=pltpu.CompilerParams(
            dimension_semantics=("arbitrary", "arbitrary"),
        ),
    )(aspect_ratio_ids, perm, gate, hs_t, embedding, tile_embedding)
    return jnp.transpose(out_t, (0, 2, 1, 3))
